# baseline (speedup 1.0000x reference)
import functools

import jax
import jax.numpy as jnp
import numpy as np
from jax import lax
from jax.experimental import pallas as pl
from jax.experimental.pallas import tpu as pltpu
from jax.experimental.pallas import tpu_sc as plsc

_B, _N, _D = 16, 100000, 6
_K = 8192
_NC, _NS = 2, 16
_NW = _NC * _NS
_NPLANES = _D * _B
_PPW = _NPLANES // _NW
_L = 16
_NG = _K // _L


def _perm_idx():
    perm = jax.random.permutation(jax.random.key(42), _N)
    return perm[:_K].astype(jnp.int32)


try:
    with jax.default_device(jax.local_devices(backend="cpu")[0]):
        _IDX = np.asarray(jax.device_get(_perm_idx()))
except Exception:
    _IDX = None


def _sc_gather(points_t, idx):
    mesh = plsc.VectorSubcoreMesh(core_axis_name="c", subcore_axis_name="s")

    @functools.partial(
        pl.kernel,
        out_type=jax.ShapeDtypeStruct((_D, _B, _K), jnp.float32),
        mesh=mesh,
        scratch_types=[
            pltpu.VMEM((_N,), jnp.float32),
            pltpu.VMEM((_K,), jnp.float32),
            pltpu.VMEM((_K,), jnp.int32),
            pltpu.SemaphoreType.DMA,
        ],
        compiler_params=pltpu.CompilerParams(needs_layout_passes=False),
    )
    def run(pts_hbm, idx_hbm, out_hbm, plane_v, out_v, idx_v, sem):
        wid = lax.axis_index("s") * _NC + lax.axis_index("c")
        pltpu.sync_copy(idx_hbm, idx_v)
        for i in range(_PPW):
            p = wid * _PPW + i
            c = p // _B
            b = p % _B
            pltpu.sync_copy(pts_hbm.at[c, b], plane_v)

            def body(g, _):
                ids = idx_v[pl.ds(g * _L, _L)]
                out_v[pl.ds(g * _L, _L)] = plsc.load_gather(plane_v, [ids])
                return _

            lax.fori_loop(0, _NG, body, 0, unroll=8)
            pltpu.sync_copy(out_v, out_hbm.at[c, b])

    return run(points_t, idx)


def kernel(points):
    idx = jnp.asarray(_IDX) if _IDX is not None else _perm_idx()
    points_t = jnp.transpose(points, (2, 0, 1))
    out_t = _sc_gather(points_t, idx)
    return jnp.transpose(out_t, (1, 2, 0))


# speedup vs baseline: 1.0330x; 1.0330x over previous
import functools

import jax
import jax.numpy as jnp
import numpy as np
from jax import lax
from jax.experimental import pallas as pl
from jax.experimental.pallas import tpu as pltpu
from jax.experimental.pallas import tpu_sc as plsc

_B, _N, _D = 16, 100000, 6
_K = 8192
_NC, _NS = 2, 16
_NW = _NC * _NS
_NPLANES = _D * _B
_PPW = _NPLANES // _NW
_L = 16
_NG = _K // _L
_QB = (0, 25088, 50176, 75264, _N)
_NQ = len(_QB) - 1
_NALN = 99968
_TAILSTART = _N - 128
_TAIL = 128


def _perm_idx():
    perm = jax.random.permutation(jax.random.key(42), _N)
    return perm[:_K].astype(jnp.int32)


try:
    with jax.default_device(jax.local_devices(backend="cpu")[0]):
        _IDX = np.asarray(jax.device_get(_perm_idx()))
except Exception:
    _IDX = None


def _partition_by_quarter(idx):
    srcs, dsts, offs, trips = [], [], [], []
    off = 0
    for q in range(_NQ):
        sel = np.where((idx >= _QB[q]) & (idx < _QB[q + 1]))[0].astype(np.int32)
        src = idx[sel]
        pad = (-len(sel)) % _L
        src = np.concatenate([src, np.full(pad, _QB[q], np.int32)])
        dst = np.concatenate([sel, (_K + np.arange(pad)).astype(np.int32)])
        srcs.append(src)
        dsts.append(dst)
        offs.append(off)
        trips.append(len(src) // _L)
        off += len(src)
    return (np.concatenate(srcs), np.concatenate(dsts), tuple(offs),
            tuple(trips))


_PART = _partition_by_quarter(_IDX) if _IDX is not None else None


def _sc_gather_pipelined(points_t, tail, src, dst, offs, trips):
    mesh = plsc.VectorSubcoreMesh(core_axis_name="c", subcore_axis_name="s")
    tot = src.shape[0]

    @functools.partial(
        pl.kernel,
        out_type=jax.ShapeDtypeStruct((_D, _B, _K), jnp.float32),
        mesh=mesh,
        scratch_types=[
            pltpu.VMEM((_N,), jnp.float32),
            pltpu.VMEM((_K + _L,), jnp.float32),
            pltpu.VMEM((tot,), jnp.int32),
            pltpu.VMEM((tot,), jnp.int32),
            pltpu.SemaphoreType.DMA,
        ],
        compiler_params=pltpu.CompilerParams(needs_layout_passes=False),
    )
    def run(pts_hbm, tail_hbm, src_hbm, dst_hbm, out_hbm, plane_v, out_v,
            src_v, dst_v, sem):
        wid = lax.axis_index("s") * _NC + lax.axis_index("c")
        pltpu.sync_copy(src_hbm, src_v)
        pltpu.sync_copy(dst_hbm, dst_v)
        for i in range(_PPW):
            p = wid * _PPW + i
            c = p // _B
            b = p % _B
            spans = [(_QB[q], min(_QB[q + 1], _NALN) - _QB[q])
                     for q in range(_NQ)]
            cps = [
                pltpu.async_copy(
                    pts_hbm.at[c, b, pl.ds(s, n)],
                    plane_v.at[pl.ds(s, n)],
                    sem,
                )
                for s, n in spans
            ]
            tail_cp = pltpu.async_copy(
                tail_hbm.at[pl.ds(pl.multiple_of(p * _TAIL, _TAIL), _TAIL)],
                plane_v.at[pl.ds(_TAILSTART, _TAIL)],
                sem,
            )
            for q in range(_NQ):
                cps[q].wait()
                if q == _NQ - 1:
                    tail_cp.wait()
                off = offs[q]

                def body(g, _, off=off):
                    at = pl.ds(off + g * _L, _L)
                    vals = plsc.load_gather(plane_v, [src_v[at]])
                    plsc.store_scatter(out_v, [dst_v[at]], vals)
                    return _

                lax.fori_loop(0, trips[q], body, 0)
            pltpu.sync_copy(out_v.at[pl.ds(0, _K)], out_hbm.at[c, b])

    return run(points_t, tail, src, dst)


def _sc_gather_uniform(points_t, idx):
    mesh = plsc.VectorSubcoreMesh(core_axis_name="c", subcore_axis_name="s")

    @functools.partial(
        pl.kernel,
        out_type=jax.ShapeDtypeStruct((_D, _B, _K), jnp.float32),
        mesh=mesh,
        scratch_types=[
            pltpu.VMEM((_N,), jnp.float32),
            pltpu.VMEM((_K,), jnp.float32),
            pltpu.VMEM((_K,), jnp.int32),
            pltpu.SemaphoreType.DMA,
        ],
        compiler_params=pltpu.CompilerParams(needs_layout_passes=False),
    )
    def run(pts_hbm, idx_hbm, out_hbm, plane_v, out_v, idx_v, sem):
        wid = lax.axis_index("s") * _NC + lax.axis_index("c")
        pltpu.sync_copy(idx_hbm, idx_v)
        for i in range(_PPW):
            p = wid * _PPW + i
            c = p // _B
            b = p % _B
            pltpu.sync_copy(pts_hbm.at[c, b], plane_v)

            def body(g, _):
                at = pl.ds(g * _L, _L)
                out_v[at] = plsc.load_gather(plane_v, [idx_v[at]])
                return _

            lax.fori_loop(0, _NG, body, 0)
            pltpu.sync_copy(out_v, out_hbm.at[c, b])

    return run(points_t, idx)


def kernel(points):
    points_t = jnp.transpose(points, (2, 0, 1))
    if _PART is not None:
        src, dst, offs, trips = _PART
        tail = jnp.reshape(
            lax.slice(points_t, (0, 0, _TAILSTART), (_D, _B, _N)),
            (_D * _B * _TAIL,))
        out_t = _sc_gather_pipelined(points_t, tail, jnp.asarray(src),
                                     jnp.asarray(dst), offs, trips)
    else:
        out_t = _sc_gather_uniform(points_t, _perm_idx())
    return jnp.transpose(out_t, (1, 2, 0))


# speedup vs baseline: 1.1806x; 1.1429x over previous
import functools

import jax
import jax.numpy as jnp
import numpy as np
from jax import lax
from jax.experimental import pallas as pl
from jax.experimental.pallas import tpu as pltpu
from jax.experimental.pallas import tpu_sc as plsc

_B, _N, _D = 16, 100000, 6
_K = 8192
_NC, _NS = 2, 16
_NW = _NC * _NS
_NPLANES = _D * _B
_PPW = _NPLANES // _NW
_L = 16
_NG = _K // _L
_QB = (0, 25088, 50176, 75264, _N)
_NQ = len(_QB) - 1
_NALN = 99968
_TAILSTART = _N - 128
_TAIL = 128


def _perm_idx():
    perm = jax.random.permutation(jax.random.key(42), _N)
    return perm[:_K].astype(jnp.int32)


try:
    with jax.default_device(jax.local_devices(backend="cpu")[0]):
        _IDX = np.asarray(jax.device_get(_perm_idx()))
except Exception:
    _IDX = None


def _partition_by_quarter(idx):
    srcs, dsts, offs, trips = [], [], [], []
    off = 0
    for q in range(_NQ):
        sel = np.where((idx >= _QB[q]) & (idx < _QB[q + 1]))[0].astype(np.int32)
        src = idx[sel]
        pad = (-len(sel)) % _L
        src = np.concatenate([src, np.full(pad, _QB[q], np.int32)])
        dst = np.concatenate([sel, (_K + np.arange(pad)).astype(np.int32)])
        srcs.append(src)
        dsts.append(dst)
        offs.append(off)
        trips.append(len(src) // _L)
        off += len(src)
    return (np.concatenate(srcs), np.concatenate(dsts), tuple(offs),
            tuple(trips))


_PART = _partition_by_quarter(_IDX) if _IDX is not None else None


def _sc_gather_pipelined(points_t, tail, src, dst, offs, trips):
    mesh = plsc.VectorSubcoreMesh(core_axis_name="c", subcore_axis_name="s")
    tot = src.shape[0]

    @functools.partial(
        pl.kernel,
        out_type=jax.ShapeDtypeStruct((_D, _B, _K), jnp.float32),
        mesh=mesh,
        scratch_types=[
            pltpu.VMEM((_N,), jnp.float32),
            pltpu.VMEM((_K + _L,), jnp.float32),
            pltpu.VMEM((tot,), jnp.int32),
            pltpu.VMEM((tot,), jnp.int32),
            pltpu.SemaphoreType.DMA,
        ],
        compiler_params=pltpu.CompilerParams(needs_layout_passes=False),
    )
    def run(pts_hbm, tail_hbm, src_hbm, dst_hbm, out_hbm, plane_v, out_v,
            src_v, dst_v, sem):
        wid = lax.axis_index("s") * _NC + lax.axis_index("c")
        pltpu.sync_copy(src_hbm, src_v)
        pltpu.sync_copy(dst_hbm, dst_v)
        for i in range(_PPW):
            p = wid * _PPW + i
            c = p // _B
            b = p % _B
            spans = [(_QB[q], min(_QB[q + 1], _NALN) - _QB[q])
                     for q in range(_NQ)]
            cps = [
                pltpu.async_copy(
                    pts_hbm.at[c, b, pl.ds(s, n)],
                    plane_v.at[pl.ds(s, n)],
                    sem,
                )
                for s, n in spans
            ]
            tail_cp = pltpu.async_copy(
                tail_hbm.at[pl.ds(pl.multiple_of(p * _TAIL, _TAIL), _TAIL)],
                plane_v.at[pl.ds(_TAILSTART, _TAIL)],
                sem,
            )
            for q in range(_NQ):
                cps[q].wait()
                if q == _NQ - 1:
                    tail_cp.wait()
                off = offs[q]

                def body(g, off=off):
                    at = pl.ds(off + g * _L, _L)
                    vals = plsc.load_gather(plane_v, [src_v[at]])
                    plsc.store_scatter(out_v, [dst_v[at]], vals)

                plsc.parallel_loop(0, trips[q], 1, unroll=4)(body)
            pltpu.sync_copy(out_v.at[pl.ds(0, _K)], out_hbm.at[c, b])

    return run(points_t, tail, src, dst)


def _sc_gather_uniform(points_t, idx):
    mesh = plsc.VectorSubcoreMesh(core_axis_name="c", subcore_axis_name="s")

    @functools.partial(
        pl.kernel,
        out_type=jax.ShapeDtypeStruct((_D, _B, _K), jnp.float32),
        mesh=mesh,
        scratch_types=[
            pltpu.VMEM((_N,), jnp.float32),
            pltpu.VMEM((_K,), jnp.float32),
            pltpu.VMEM((_K,), jnp.int32),
            pltpu.SemaphoreType.DMA,
        ],
        compiler_params=pltpu.CompilerParams(needs_layout_passes=False),
    )
    def run(pts_hbm, idx_hbm, out_hbm, plane_v, out_v, idx_v, sem):
        wid = lax.axis_index("s") * _NC + lax.axis_index("c")
        pltpu.sync_copy(idx_hbm, idx_v)
        for i in range(_PPW):
            p = wid * _PPW + i
            c = p // _B
            b = p % _B
            pltpu.sync_copy(pts_hbm.at[c, b], plane_v)

            def body(g, _):
                at = pl.ds(g * _L, _L)
                out_v[at] = plsc.load_gather(plane_v, [idx_v[at]])
                return _

            lax.fori_loop(0, _NG, body, 0)
            pltpu.sync_copy(out_v, out_hbm.at[c, b])

    return run(points_t, idx)


def kernel(points):
    points_t = jnp.transpose(points, (2, 0, 1))
    if _PART is not None:
        src, dst, offs, trips = _PART
        tail = jnp.reshape(
            lax.slice(points_t, (0, 0, _TAILSTART), (_D, _B, _N)),
            (_D * _B * _TAIL,))
        out_t = _sc_gather_pipelined(points_t, tail, jnp.asarray(src),
                                     jnp.asarray(dst), offs, trips)
    else:
        out_t = _sc_gather_uniform(points_t, _perm_idx())
    return jnp.transpose(out_t, (1, 2, 0))


# speedup vs baseline: 1.1855x; 1.0041x over previous
import functools

import jax
import jax.numpy as jnp
import numpy as np
from jax import lax
from jax.experimental import pallas as pl
from jax.experimental.pallas import tpu as pltpu
from jax.experimental.pallas import tpu_sc as plsc

_B, _N, _D = 16, 100000, 6
_K = 8192
_NC, _NS = 2, 16
_NW = _NC * _NS
_NPLANES = _D * _B
_PPW = _NPLANES // _NW
_L = 16
_NG = _K // _L
_QB = (0, 25088, 50176, 75264, _N)
_NQ = len(_QB) - 1
_NALN = 99968
_TAILSTART = _N - 128
_TAIL = 128


def _perm_idx():
    perm = jax.random.permutation(jax.random.key(42), _N)
    return perm[:_K].astype(jnp.int32)


try:
    with jax.default_device(jax.local_devices(backend="cpu")[0]):
        _IDX = np.asarray(jax.device_get(_perm_idx()))
except Exception:
    _IDX = None


def _partition_by_quarter(idx):
    srcs, dsts, offs, trips = [], [], [], []
    off = 0
    for q in range(_NQ):
        sel = np.where((idx >= _QB[q]) & (idx < _QB[q + 1]))[0].astype(np.int32)
        src = idx[sel]
        pad = (-len(sel)) % _L
        src = np.concatenate([src, np.full(pad, _QB[q], np.int32)])
        dst = np.concatenate([sel, (_K + np.arange(pad)).astype(np.int32)])
        srcs.append(src)
        dsts.append(dst)
        offs.append(off)
        trips.append(len(src) // _L)
        off += len(src)
    return (np.concatenate(srcs), np.concatenate(dsts), tuple(offs),
            tuple(trips))


_PART = _partition_by_quarter(_IDX) if _IDX is not None else None


def _sc_gather_pipelined(points_t, tail, src, dst, offs, trips):
    mesh = plsc.VectorSubcoreMesh(core_axis_name="c", subcore_axis_name="s")
    tot = src.shape[0]

    @functools.partial(
        pl.kernel,
        out_type=jax.ShapeDtypeStruct((_D, _B, _K), jnp.float32),
        mesh=mesh,
        scratch_types=[
            pltpu.VMEM((_N,), jnp.float32),
            pltpu.VMEM((_K + _L,), jnp.float32),
            pltpu.VMEM((tot,), jnp.int32),
            pltpu.VMEM((tot,), jnp.int32),
            pltpu.SemaphoreType.DMA,
        ],
        compiler_params=pltpu.CompilerParams(needs_layout_passes=False),
    )
    def run(pts_hbm, tail_hbm, src_hbm, dst_hbm, out_hbm, plane_v, out_v,
            src_v, dst_v, sem):
        wid = lax.axis_index("s") * _NC + lax.axis_index("c")
        pltpu.sync_copy(src_hbm, src_v)
        pltpu.sync_copy(dst_hbm, dst_v)
        def plane_body(i, _):
            p = wid * _PPW + i
            c = p // _B
            b = p % _B
            spans = [(_QB[q], min(_QB[q + 1], _NALN) - _QB[q])
                     for q in range(_NQ)]
            cps = [
                pltpu.async_copy(
                    pts_hbm.at[c, b, pl.ds(s, n)],
                    plane_v.at[pl.ds(s, n)],
                    sem,
                )
                for s, n in spans
            ]
            tail_cp = pltpu.async_copy(
                tail_hbm.at[pl.ds(pl.multiple_of(p * _TAIL, _TAIL), _TAIL)],
                plane_v.at[pl.ds(_TAILSTART, _TAIL)],
                sem,
            )
            for q in range(_NQ):
                cps[q].wait()
                if q == _NQ - 1:
                    tail_cp.wait()
                off = offs[q]

                def body(g, off=off):
                    at = pl.ds(off + g * _L, _L)
                    vals = plsc.load_gather(plane_v, [src_v[at]])
                    plsc.store_scatter(out_v, [dst_v[at]], vals)

                plsc.parallel_loop(0, trips[q], 1, unroll=8)(body)
            pltpu.sync_copy(out_v.at[pl.ds(0, _K)], out_hbm.at[c, b])
            return _

        lax.fori_loop(0, _PPW, plane_body, 0)

    return run(points_t, tail, src, dst)


def _sc_gather_uniform(points_t, idx):
    mesh = plsc.VectorSubcoreMesh(core_axis_name="c", subcore_axis_name="s")

    @functools.partial(
        pl.kernel,
        out_type=jax.ShapeDtypeStruct((_D, _B, _K), jnp.float32),
        mesh=mesh,
        scratch_types=[
            pltpu.VMEM((_N,), jnp.float32),
            pltpu.VMEM((_K,), jnp.float32),
            pltpu.VMEM((_K,), jnp.int32),
            pltpu.SemaphoreType.DMA,
        ],
        compiler_params=pltpu.CompilerParams(needs_layout_passes=False),
    )
    def run(pts_hbm, idx_hbm, out_hbm, plane_v, out_v, idx_v, sem):
        wid = lax.axis_index("s") * _NC + lax.axis_index("c")
        pltpu.sync_copy(idx_hbm, idx_v)
        for i in range(_PPW):
            p = wid * _PPW + i
            c = p // _B
            b = p % _B
            pltpu.sync_copy(pts_hbm.at[c, b], plane_v)

            def body(g, _):
                at = pl.ds(g * _L, _L)
                out_v[at] = plsc.load_gather(plane_v, [idx_v[at]])
                return _

            lax.fori_loop(0, _NG, body, 0)
            pltpu.sync_copy(out_v, out_hbm.at[c, b])

    return run(points_t, idx)


def kernel(points):
    points_t = jnp.transpose(points, (2, 0, 1))
    if _PART is not None:
        src, dst, offs, trips = _PART
        tail = jnp.reshape(
            lax.slice(points_t, (0, 0, _TAILSTART), (_D, _B, _N)),
            (_D * _B * _TAIL,))
        out_t = _sc_gather_pipelined(points_t, tail, jnp.asarray(src),
                                     jnp.asarray(dst), offs, trips)
    else:
        out_t = _sc_gather_uniform(points_t, _perm_idx())
    return jnp.transpose(out_t, (1, 2, 0))
